# baseline (speedup 1.0000x reference)
import functools

import jax
import jax.numpy as jnp
from jax import lax
from jax.experimental import pallas as pl
from jax.experimental.pallas import tpu as pltpu
from jax.experimental.pallas import tpu_sc as plsc

_B = 4096
_D = 32
_R = 64
_RD = _R * _D
_NC = 2
_NS = 16
_NW = _NC * _NS
_BPW = _B // _NW
_N = 1000000
_HC = 8
_NH = _BPW // _HC

_BB = 512
_G = _B // _BB


def _sc_gather(users, pos_items, neg_items, heads, pos_tails, neg_tails,
               ueT, ieT, keT):
    mesh = plsc.VectorSubcoreMesh(core_axis_name="c", subcore_axis_name="s")
    out_t = [jax.ShapeDtypeStruct((_D, _B), jnp.float32)] * 8

    @functools.partial(
        pl.kernel,
        mesh=mesh,
        out_type=out_t,
        compiler_params=pltpu.CompilerParams(needs_layout_passes=False),
        scratch_types=[
            pltpu.VMEM((2, _BPW + 16), jnp.int32),
            pltpu.VMEM((3, _HC, _D, 128), jnp.float32),
            pltpu.VMEM((2, _D, _BPW), jnp.float32),
            pltpu.SemaphoreType.DMA,
            pltpu.SemaphoreType.DMA,
        ],
    )
    def k(users_h, pos_h, neg_h, heads_h, pt_h, nt_h,
          ue_h, ie_h, ke_h,
          u_o, pi_o, pkg_o, ni_o, nkg_o, h_o, pt_o, nt_o,
          idx_v, pan_v, fbuf, sem, ssem):
        wid = lax.axis_index("s") * _NC + lax.axis_index("c")
        base = wid * _BPW
        tabs = (
            (0, users_h, ue_h, u_o),
            (1, pos_h, ie_h, pi_o),
            (1, pos_h, ke_h, pkg_o),
            (2, neg_h, ie_h, ni_o),
            (2, neg_h, ke_h, nkg_o),
            (3, heads_h, ke_h, h_o),
            (4, pt_h, ke_h, pt_o),
            (5, nt_h, ke_h, nt_o),
        )
        f16a = lax.iota(jnp.int32, 16)
        f16b = f16a + 16

        def fire(tab_h, js, ph, h):
            slot = lax.rem(h + ph, 3)
            v = idx_v[js, pl.ds(h * _HC, 16)]
            for kk in range(_HC):
                po = pl.multiple_of((v[kk] >> 7) << 7, 128)
                pltpu.async_copy(tab_h.at[:, pl.ds(po, 128)],
                                 pan_v.at[slot, kk], sem)

        def drain_extract(tab_h, js, ph, h, fp):
            slot = lax.rem(h + ph, 3)
            for kk in range(_HC):
                pltpu.make_async_copy(tab_h.at[:, pl.ds(0, 128)],
                                      pan_v.at[slot, kk], sem).wait()
            v = idx_v[js, pl.ds(h * _HC, 16)]
            for kk in range(_HC):
                c = jnp.full((16,), v[kk] & 127, jnp.int32)
                j = jnp.full((16,), h * _HC + kk, jnp.int32)
                va = plsc.load_gather(pan_v.at[slot, kk], [f16a, c])
                vb = plsc.load_gather(pan_v.at[slot, kk], [f16b, c])
                plsc.store_scatter(fbuf.at[fp], [f16a, j], va)
                plsc.store_scatter(fbuf.at[fp], [f16b, j], vb)

        pltpu.sync_copy(users_h.at[pl.ds(base, _BPW)],
                        idx_v.at[0, pl.ds(0, _BPW)])
        fire(ue_h, 0, 0, jnp.int32(0))
        fire(ue_h, 0, 0, jnp.int32(1))

        for i, (jb, idx_h, tab_h, out_h) in enumerate(tabs):
            js = jb % 2
            ph = (i * _NH) % 3
            fp = i % 2
            if i >= 2:
                pltpu.make_async_copy(
                    fbuf.at[fp],
                    tabs[i - 2][3].at[:, pl.ds(base, _BPW)], ssem).wait()

            def body(h, _, tab_h=tab_h, js=js, ph=ph, fp=fp):
                fire(tab_h, js, ph, h + 2)
                drain_extract(tab_h, js, ph, h, fp)
                return 0

            lax.fori_loop(0, _NH - 2, body, 0)
            if i + 1 < len(tabs):
                njb, nidx_h, ntab_h, _no = tabs[i + 1]
                njs = njb % 2
                nph = ((i + 1) * _NH) % 3
                if njb != jb:
                    pltpu.sync_copy(nidx_h.at[pl.ds(base, _BPW)],
                                    idx_v.at[njs, pl.ds(0, _BPW)])
                fire(ntab_h, njs, nph, jnp.int32(0))
                drain_extract(tab_h, js, ph, jnp.int32(_NH - 2), fp)
                fire(ntab_h, njs, nph, jnp.int32(1))
                drain_extract(tab_h, js, ph, jnp.int32(_NH - 1), fp)
            else:
                drain_extract(tab_h, js, ph, jnp.int32(_NH - 2), fp)
                drain_extract(tab_h, js, ph, jnp.int32(_NH - 1), fp)
            pltpu.async_copy(fbuf.at[fp], out_h.at[:, pl.ds(base, _BPW)],
                             ssem)

        for i in (len(tabs) - 2, len(tabs) - 1):
            pltpu.make_async_copy(
                fbuf.at[i % 2],
                tabs[i][3].at[:, pl.ds(base, _BPW)], ssem).wait()

    return k(users, pos_items, neg_items, heads, pos_tails, neg_tails,
             ueT, ieT, keT)


def _l2nT(x):
    n = jnp.sqrt(jnp.sum(x * x, axis=0, keepdims=True))
    return x / jnp.maximum(n, 1e-12)


def _tc_body(u_ref, pif_ref, pkgf_ref, pi_ref, pkg_ref, ni_ref, nkg_ref,
             h_ref, ptr_ref, ntr_ref, krel_ref, rel_ref, w2_ref,
             pc_ref, nc_ref, hn_ref, rn_ref, ptn_ref, ntn_ref, pred_ref):
    comb_fullT = pif_ref[...] + pkgf_ref[...]
    pc_ref[...] = pi_ref[...] + pkg_ref[...]
    nc_ref[...] = ni_ref[...] + nkg_ref[...]

    pred_ref[...] = lax.dot_general(
        u_ref[...], comb_fullT,
        dimension_numbers=(((0,), (0,)), ((), ())),
        preferred_element_type=jnp.float32)

    relrow = rel_ref[0:1, :]
    subl = lax.broadcasted_iota(jnp.int32, (_RD, _BB), 0)
    oh_expT = (jnp.broadcast_to(relrow, (_RD, _BB)) ==
               (subl // _D)).astype(jnp.float32)

    iota_r = lax.broadcasted_iota(jnp.int32, (_R, _BB), 0)
    oh_rT = (jnp.broadcast_to(relrow, (_R, _BB)) ==
             iota_r).astype(jnp.float32)
    rn_ref[...] = _l2nT(jnp.dot(krel_ref[...], oh_rT,
                                preferred_element_type=jnp.float32))

    ic = lax.broadcasted_iota(jnp.int32, (_D, _RD), 1)
    io = lax.broadcasted_iota(jnp.int32, (_D, _RD), 0)
    selT = ((ic % _D) == io).astype(jnp.float32)

    w2T = w2_ref[...]
    for xT_ref, outT_ref in ((h_ref, hn_ref), (ptr_ref, ptn_ref),
                             (ntr_ref, ntn_ref)):
        yT = jnp.dot(w2T, xT_ref[...],
                     preferred_element_type=jnp.float32)
        projT = jnp.dot(selT, yT * oh_expT,
                        preferred_element_type=jnp.float32)
        outT_ref[...] = _l2nT(projT)


def _tc_dense(uT, piT, pkgT, niT, nkgT, hT, ptT, ntT, krelT, rel8, w2T):
    blk = pl.BlockSpec((_D, _BB), lambda i: (0, i))
    full = pl.BlockSpec((_D, _B), lambda i: (0, 0))
    small = jax.ShapeDtypeStruct((_D, _B), jnp.float32)
    return pl.pallas_call(
        _tc_body,
        grid=(_G,),
        in_specs=[
            blk,
            full, full,
            blk, blk,
            blk, blk,
            blk, blk, blk,
            pl.BlockSpec((_D, _R), lambda i: (0, 0)),
            pl.BlockSpec((8, _BB), lambda i: (0, i)),
            pl.BlockSpec((_RD, _D), lambda i: (0, 0)),
        ],
        out_specs=[
            blk, blk, blk, blk, blk, blk,
            pl.BlockSpec((_BB, _B), lambda i: (i, 0)),
        ],
        out_shape=[small] * 6 + [
            jax.ShapeDtypeStruct((_B, _B), jnp.float32),
        ],
    )(uT, piT, pkgT, piT, pkgT, niT, nkgT, hT, ptT, ntT, krelT, rel8, w2T)


def kernel(users, pos_items, neg_items, heads, relations, pos_tails, neg_tails,
           user_embed, item_embed, kg_entity_embed, kg_relation_embed,
           trans_W):
    ueT = user_embed.T
    ieT = item_embed.T
    keT = kg_entity_embed.T
    uT, piT, pkgT, niT, nkgT, hT, ptT, ntT = _sc_gather(
        users, pos_items, neg_items, heads, pos_tails, neg_tails,
        ueT, ieT, keT)
    w2T = jnp.transpose(trans_W, (0, 2, 1)).reshape(_RD, _D)
    krelT = kg_relation_embed.T
    rel8 = jnp.broadcast_to(relations.reshape(1, _B), (8, _B))
    pcT, ncT, hnT, rnT, ptnT, ntnT, preds = _tc_dense(
        uT, piT, pkgT, niT, nkgT, hT, ptT, ntT, krelT, rel8, w2T)
    return (uT.T, pcT.T, ncT.T, hnT.T, rnT.T, ptnT.T, ntnT.T, preds)


# speedup vs baseline: 1.0089x; 1.0089x over previous
import functools

import jax
import jax.numpy as jnp
from jax import lax
from jax.experimental import pallas as pl
from jax.experimental.pallas import tpu as pltpu
from jax.experimental.pallas import tpu_sc as plsc

_B = 4096
_D = 32
_R = 64
_RD = _R * _D
_NC = 2
_NS = 16
_NW = _NC * _NS
_BPW = _B // _NW
_N = 1000000
_HC = 8
_NH = _BPW // _HC

_BB = 512
_G = _B // _BB


def _sc_gather(users, pos_items, neg_items, heads, pos_tails, neg_tails,
               ueT, ieT, keT):
    mesh = plsc.VectorSubcoreMesh(core_axis_name="c", subcore_axis_name="s")
    out_t = [jax.ShapeDtypeStruct((_D, _B), jnp.float32)] * 8

    @functools.partial(
        pl.kernel,
        mesh=mesh,
        out_type=out_t,
        compiler_params=pltpu.CompilerParams(needs_layout_passes=False),
        scratch_types=[
            pltpu.VMEM((2, _BPW + 16), jnp.int32),
            pltpu.VMEM((3, _HC, _D, 128), jnp.float32),
            pltpu.VMEM((_D, _BPW), jnp.float32),
            pltpu.SemaphoreType.DMA,
        ],
    )
    def k(users_h, pos_h, neg_h, heads_h, pt_h, nt_h,
          ue_h, ie_h, ke_h,
          u_o, pi_o, pkg_o, ni_o, nkg_o, h_o, pt_o, nt_o,
          idx_v, pan_v, fbuf, sem):
        wid = lax.axis_index("s") * _NC + lax.axis_index("c")
        base = wid * _BPW
        tabs = (
            (0, users_h, ue_h, u_o),
            (1, pos_h, ie_h, pi_o),
            (1, pos_h, ke_h, pkg_o),
            (2, neg_h, ie_h, ni_o),
            (2, neg_h, ke_h, nkg_o),
            (3, heads_h, ke_h, h_o),
            (4, pt_h, ke_h, pt_o),
            (5, nt_h, ke_h, nt_o),
        )
        f16a = lax.iota(jnp.int32, 16)
        f16b = f16a + 16

        def fire(tab_h, js, ph, h):
            slot = lax.rem(h + ph, 3)
            v = idx_v[js, pl.ds(h * _HC, 16)]
            for kk in range(_HC):
                po = pl.multiple_of((v[kk] >> 7) << 7, 128)
                pltpu.async_copy(tab_h.at[:, pl.ds(po, 128)],
                                 pan_v.at[slot, kk], sem)

        def drain_extract(tab_h, js, ph, h):
            slot = lax.rem(h + ph, 3)
            for kk in range(_HC):
                pltpu.make_async_copy(tab_h.at[:, pl.ds(0, 128)],
                                      pan_v.at[slot, kk], sem).wait()
            v = idx_v[js, pl.ds(h * _HC, 16)]
            for kk in range(_HC):
                c = jnp.full((16,), v[kk] & 127, jnp.int32)
                j = jnp.full((16,), h * _HC + kk, jnp.int32)
                va = plsc.load_gather(pan_v.at[slot, kk], [f16a, c])
                vb = plsc.load_gather(pan_v.at[slot, kk], [f16b, c])
                plsc.store_scatter(fbuf, [f16a, j], va)
                plsc.store_scatter(fbuf, [f16b, j], vb)

        pltpu.sync_copy(users_h.at[pl.ds(base, _BPW)],
                        idx_v.at[0, pl.ds(0, _BPW)])
        fire(ue_h, 0, 0, jnp.int32(0))
        fire(ue_h, 0, 0, jnp.int32(1))

        for i, (jb, idx_h, tab_h, out_h) in enumerate(tabs):
            js = jb % 2
            ph = (i * _NH) % 3

            def body(h, _, tab_h=tab_h, js=js, ph=ph):
                fire(tab_h, js, ph, h + 2)
                drain_extract(tab_h, js, ph, h)
                return 0

            lax.fori_loop(0, _NH - 2, body, 0)
            if i + 1 < len(tabs):
                njb, nidx_h, ntab_h, _no = tabs[i + 1]
                njs = njb % 2
                nph = ((i + 1) * _NH) % 3
                if njb != jb:
                    pltpu.sync_copy(nidx_h.at[pl.ds(base, _BPW)],
                                    idx_v.at[njs, pl.ds(0, _BPW)])
                fire(ntab_h, njs, nph, jnp.int32(0))
                drain_extract(tab_h, js, ph, jnp.int32(_NH - 2))
                fire(ntab_h, njs, nph, jnp.int32(1))
                drain_extract(tab_h, js, ph, jnp.int32(_NH - 1))
            else:
                drain_extract(tab_h, js, ph, jnp.int32(_NH - 2))
                drain_extract(tab_h, js, ph, jnp.int32(_NH - 1))
            pltpu.sync_copy(fbuf, out_h.at[:, pl.ds(base, _BPW)])

    return k(users, pos_items, neg_items, heads, pos_tails, neg_tails,
             ueT, ieT, keT)


def _l2nT(x):
    n = jnp.sqrt(jnp.sum(x * x, axis=0, keepdims=True))
    return x / jnp.maximum(n, 1e-12)


def _tc_body(u_ref, pif_ref, pkgf_ref, pi_ref, pkg_ref, ni_ref, nkg_ref,
             h_ref, ptr_ref, ntr_ref, krel_ref, rel_ref, w2_ref,
             pc_ref, nc_ref, hn_ref, rn_ref, ptn_ref, ntn_ref, pred_ref):
    comb_fullT = pif_ref[...] + pkgf_ref[...]
    pc_ref[...] = pi_ref[...] + pkg_ref[...]
    nc_ref[...] = ni_ref[...] + nkg_ref[...]

    pred_ref[...] = lax.dot_general(
        u_ref[...], comb_fullT,
        dimension_numbers=(((0,), (0,)), ((), ())),
        preferred_element_type=jnp.float32)

    relrow = rel_ref[0:1, :]
    subl = lax.broadcasted_iota(jnp.int32, (_RD, _BB), 0)
    oh_expT = (jnp.broadcast_to(relrow, (_RD, _BB)) ==
               (subl // _D)).astype(jnp.float32)

    iota_r = lax.broadcasted_iota(jnp.int32, (_R, _BB), 0)
    oh_rT = (jnp.broadcast_to(relrow, (_R, _BB)) ==
             iota_r).astype(jnp.float32)
    rn_ref[...] = _l2nT(jnp.dot(krel_ref[...], oh_rT,
                                preferred_element_type=jnp.float32))

    ic = lax.broadcasted_iota(jnp.int32, (_D, _RD), 1)
    io = lax.broadcasted_iota(jnp.int32, (_D, _RD), 0)
    selT = ((ic % _D) == io).astype(jnp.float32)

    w2T = w2_ref[...]
    for xT_ref, outT_ref in ((h_ref, hn_ref), (ptr_ref, ptn_ref),
                             (ntr_ref, ntn_ref)):
        yT = jnp.dot(w2T, xT_ref[...],
                     preferred_element_type=jnp.float32)
        projT = jnp.dot(selT, yT * oh_expT,
                        preferred_element_type=jnp.float32)
        outT_ref[...] = _l2nT(projT)


def _tc_dense(uT, piT, pkgT, niT, nkgT, hT, ptT, ntT, krelT, rel8, w2T):
    blk = pl.BlockSpec((_D, _BB), lambda i: (0, i))
    full = pl.BlockSpec((_D, _B), lambda i: (0, 0))
    small = jax.ShapeDtypeStruct((_D, _B), jnp.float32)
    return pl.pallas_call(
        _tc_body,
        grid=(_G,),
        in_specs=[
            blk,
            full, full,
            blk, blk,
            blk, blk,
            blk, blk, blk,
            pl.BlockSpec((_D, _R), lambda i: (0, 0)),
            pl.BlockSpec((8, _BB), lambda i: (0, i)),
            pl.BlockSpec((_RD, _D), lambda i: (0, 0)),
        ],
        out_specs=[
            blk, blk, blk, blk, blk, blk,
            pl.BlockSpec((_BB, _B), lambda i: (i, 0)),
        ],
        out_shape=[small] * 6 + [
            jax.ShapeDtypeStruct((_B, _B), jnp.float32),
        ],
    )(uT, piT, pkgT, piT, pkgT, niT, nkgT, hT, ptT, ntT, krelT, rel8, w2T)


def kernel(users, pos_items, neg_items, heads, relations, pos_tails, neg_tails,
           user_embed, item_embed, kg_entity_embed, kg_relation_embed,
           trans_W):
    ueT = user_embed.T
    ieT = item_embed.T
    keT = kg_entity_embed.T
    uT, piT, pkgT, niT, nkgT, hT, ptT, ntT = _sc_gather(
        users, pos_items, neg_items, heads, pos_tails, neg_tails,
        ueT, ieT, keT)
    w2T = jnp.transpose(trans_W, (0, 2, 1)).reshape(_RD, _D)
    krelT = kg_relation_embed.T
    rel8 = jnp.broadcast_to(relations.reshape(1, _B), (8, _B))
    pcT, ncT, hnT, rnT, ptnT, ntnT, preds = _tc_dense(
        uT, piT, pkgT, niT, nkgT, hT, ptT, ntT, krelT, rel8, w2T)
    return (uT.T, pcT.T, ncT.T, hnT.T, rnT.T, ptnT.T, ntnT.T, preds)
